# baseline (speedup 1.0000x reference)
import functools

import jax
import jax.numpy as jnp
from jax import lax
from jax.experimental import pallas as pl
from jax.experimental.pallas import tpu as pltpu
from jax.experimental.pallas import tpu_sc as plsc

W = 1024
W_SHIFT = W.bit_length() - 1
assert W == 1 << W_SHIFT

_NLP = pltpu.CompilerParams(needs_layout_passes=False)


@functools.cache
def _build(v, d, n_rows, n_batch, n_titles):
    info = plsc.get_sparse_core_info()
    nc, ns = info.num_cores, info.num_subcores
    nw = nc * ns
    v_al = v - v % 128
    tail = v - v_al
    n_chunks = (v_al + W - 1) // W
    last_w = v_al - (n_chunks - 1) * W
    per_w = n_rows // nw
    assert n_rows % (nw * 16) == 0 and d == 64 and n_batch % (nw * 128) == 0
    assert last_w % 128 == 0

    mesh = plsc.VectorSubcoreMesh(core_axis_name="c", subcore_axis_name="s")
    n_inter = n_rows + 16

    @functools.partial(
        pl.kernel,
        mesh=mesh,
        compiler_params=_NLP,
        out_type=jax.ShapeDtypeStruct((n_inter, 128), jnp.float32),
        scratch_types=[
            pltpu.VMEM((per_w + 16,), jnp.int32),
            pltpu.VMEM((per_w + 16,), jnp.int32),
            pltpu.VMEM((16,), jnp.int32),
            pltpu.VMEM((d, W), jnp.float32),
            pltpu.VMEM((d, tail or 128), jnp.float32),
            pltpu.VMEM((16, 128), jnp.float32),
            pltpu.SemaphoreType.DMA,
        ],
    )
    def phase1(sr_hbm, sm_hbm, meta_hbm, table_hbm, inter_hbm,
               hr, hm, meta_v, slab, tslab, stage, sem):
        wid = lax.axis_index("s") * nc + lax.axis_index("c")
        lanes = lax.iota(jnp.int32, 16)
        pltpu.sync_copy(sr_hbm.at[pl.ds(wid * per_w, per_w + 16)], hr)
        pltpu.sync_copy(sm_hbm.at[pl.ds(wid * per_w, per_w + 16)], hm)
        pltpu.sync_copy(meta_hbm.at[wid], meta_v)
        meta_vec = meta_v[pl.ds(0, 16)]
        c_lo = meta_vec[0]
        c_cnt = meta_vec[1]

        def extract_group(buf, width, r, m, base, end):
            mask = (r >= base) & (r < end)
            rl = jnp.clip(r - base, 0, width - 1)
            for cc in range(d):
                col = jnp.full((16,), cc, jnp.int32)
                vals = plsc.load_gather(buf, [col, rl])
                plsc.store_scatter(stage, [lanes, col], vals)
            mv = jnp.where(mask, m, jnp.int32(n_rows))
            pltpu.async_copy(stage, inter_hbm.at[mv], sem).wait()

        def chunk_body(j, p):
            c = c_lo + j
            is_tail = c == n_chunks
            base = jnp.where(is_tail, v_al, c * W)
            end = jnp.where(is_tail, v, jnp.minimum(c * W + W, v_al))

            @pl.when(jnp.logical_not(is_tail) & (c != n_chunks - 1))
            def _():
                pltpu.sync_copy(table_hbm.at[:, pl.ds(c * W, W)], slab)

            @pl.when(c == n_chunks - 1)
            def _():
                pltpu.sync_copy(
                    table_hbm.at[:, pl.ds((n_chunks - 1) * W, last_w)],
                    slab if last_w == W else slab.at[:, pl.ds(0, last_w)])

            if tail:
                @pl.when(is_tail)
                def _():
                    pltpu.sync_copy(table_hbm.at[:, pl.ds(v_al, tail)], tslab)

            def cond(carry2):
                p2, rf2 = carry2
                return (p2 < per_w) & (rf2 < end)

            def body(carry2):
                p2, _ = carry2
                r = hr[pl.ds(p2, 16)]
                m = hm[pl.ds(p2, 16)]

                @pl.when(jnp.logical_not(is_tail))
                def _():
                    extract_group(slab, W, r, m, base, end)

                if tail:
                    @pl.when(is_tail)
                    def _():
                        extract_group(tslab, tail, r, m, base, end)

                adv = r[15] < end
                p3 = p2 + jnp.where(adv, 16, 0)
                rf3 = hr[pl.ds(jnp.minimum(p3, per_w), 16)][0]
                return p3, jnp.where(adv, rf3, end)

            rf = hr[pl.ds(jnp.minimum(p, per_w), 16)][0]
            p, _ = lax.while_loop(cond, body, (p, rf))
            return p

        lax.fori_loop(0, c_cnt, chunk_body, 0)

    b_per_w = n_batch // nw

    @functools.partial(
        pl.kernel,
        mesh=mesh,
        compiler_params=_NLP,
        out_type=jax.ShapeDtypeStruct((n_titles, d, n_batch), jnp.float32),
        scratch_types=[
            pltpu.VMEM((b_per_w, 128), jnp.float32),
            pltpu.VMEM((d, b_per_w), jnp.float32),
        ],
    )
    def phase2(inter_hbm, out_hbm, blk, stage):
        wid = lax.axis_index("s") * nc + lax.axis_index("c")
        lanes = lax.iota(jnp.int32, 16)

        def l_body(l, carry):
            pltpu.sync_copy(
                inter_hbm.at[pl.ds(l * n_batch + wid * b_per_w, b_per_w)], blk)
            for cc in range(d):
                colv = jnp.full((16,), cc, jnp.int32)
                for bq in range(b_per_w // 16):
                    vals = plsc.load_gather(blk, [bq * 16 + lanes, colv])
                    stage[cc, pl.ds(bq * 16, 16)] = vals
            pltpu.sync_copy(
                stage, out_hbm.at[l, :, pl.ds(wid * b_per_w, b_per_w)])
            return carry

        lax.fori_loop(0, n_titles, l_body, 0)

    return phase1, phase2


def kernel(title_text, title_mask, word_embedding):
    b, l = title_text.shape
    v, d = word_embedding.shape
    n_rows = b * l
    phase1, phase2 = _build(v, d, n_rows, b, l)
    info = plsc.get_sparse_core_info()
    nw = info.num_cores * info.num_subcores
    per_w = n_rows // nw
    v_al = v - v % 128
    n_chunks = (v_al + W - 1) // W

    idx_m = title_text.T.reshape(-1).astype(jnp.int32)
    sr, sm = lax.sort_key_val(idx_m, lax.iota(jnp.int32, n_rows))
    cid = jnp.where(sr >= v_al, n_chunks, sr >> W_SHIFT)
    c_lo = cid[::per_w]
    c_cnt = cid[per_w - 1::per_w] - c_lo + 1
    meta = jnp.zeros((nw, 16), jnp.int32)
    meta = meta.at[:, 0].set(c_lo).at[:, 1].set(c_cnt)
    pad_r = jnp.full((16,), jnp.int32(0x7FFFFFFF))
    pad_m = jnp.full((16,), jnp.int32(n_rows))
    srp = jnp.concatenate([sr, pad_r])
    smp = jnp.concatenate([sm, pad_m])

    inter = phase1(srp, smp, meta, word_embedding.T)
    out_t = phase2(inter)
    return out_t.transpose(2, 0, 1)
